# baseline (speedup 1.0000x reference)
import functools

import jax
import jax.numpy as jnp
from jax import lax
from jax.experimental import pallas as pl
from jax.experimental.pallas import tpu as pltpu
from jax.experimental.pallas import tpu_sc as plsc

VOCAB = 1000000
S = 200
D = 64
B = 4096
NC = 2
NS = 16
NW = NC * NS
BPW = B // NW
LB = D // 16

DG = 4
VG = 8
GTOT = VOCAB // 16
GCNT = 7816
CG = 100
ACH = 80
CTOK = CG * 16

_mesh = plsc.VectorSubcoreMesh(core_axis_name="c", subcore_axis_name="s")
_params = pltpu.CompilerParams(
    use_tc_tiling_on_sc=False, needs_layout_passes=False)


@functools.partial(
    pl.kernel,
    mesh=_mesh,
    compiler_params=_params,
    out_type=jax.ShapeDtypeStruct((VOCAB, D), jnp.float32),
    scratch_types=[
        pltpu.VMEM((2, 16, CTOK), jnp.float32),
        pltpu.VMEM((2, CTOK, 16), jnp.float32),
        pltpu.SemaphoreType.DMA,
        pltpu.SemaphoreType.DMA,
        pltpu.SemaphoreType.DMA,
        pltpu.SemaphoreType.DMA,
    ],
)
def _transpose_table(tokt_hbm, tokr_hbm, in_v, out_v, isem0, isem1,
                     osem0, osem1):
    wid = lax.axis_index("s") * NC + lax.axis_index("c")
    dg = wid // VG
    vg = wid % VG
    d0 = dg * 16
    start_g = jnp.minimum(vg * (GTOT // VG), GTOT - GCNT)
    isems = (isem0, isem1)
    osems = (osem0, osem1)
    iota = lax.broadcasted_iota(jnp.int32, (16,), 0)

    def v0_of(c):
        return (start_g + jnp.minimum(c * CG, GCNT - CG)) * 16

    def fire_in(c, bufi, sem):
        pltpu.async_copy(
            tokt_hbm.at[pl.ds(d0, 16), pl.ds(v0_of(c), CTOK)],
            in_v.at[bufi], sem,
        )

    def wait_in(bufi, sem):
        pltpu.make_async_copy(
            tokt_hbm.at[pl.ds(0, 16), pl.ds(0, CTOK)], in_v.at[bufi], sem,
        ).wait()

    def wait_out(bufi, sem):
        pltpu.make_async_copy(
            out_v.at[bufi], tokr_hbm.at[pl.ds(0, CTOK), pl.ds(0, 16)], sem,
        ).wait()

    fire_in(0, 0, isem0)

    def pair(i, carry):
        for bufi in range(2):
            c = 2 * i + bufi
            wait_in(bufi, isems[bufi])

            @pl.when(c + 1 <= ACH - 1)
            def _():
                fire_in(c + 1, 1 - bufi, isems[1 - bufi])

            @pl.when(c >= 2)
            def _():
                wait_out(bufi, osems[bufi])

            def tr(g, inner):
                rowi = g * 16 + iota
                for dl in range(16):
                    v = in_v[bufi, dl, pl.ds(g * 16, 16)]
                    plsc.store_scatter(
                        out_v.at[bufi], [rowi, jnp.full((16,), dl, jnp.int32)],
                        v,
                    )
                return inner

            lax.fori_loop(0, CG, tr, 0)
            pltpu.async_copy(
                out_v.at[bufi],
                tokr_hbm.at[pl.ds(v0_of(c), CTOK), pl.ds(d0, 16)],
                osems[bufi],
            )
        return carry

    lax.fori_loop(0, ACH // 2, pair, 0)
    wait_out(0, osem0)
    wait_out(1, osem1)


@functools.partial(
    pl.kernel,
    mesh=_mesh,
    compiler_params=_params,
    out_type=jax.ShapeDtypeStruct((S, D, B), jnp.float32),
    scratch_types=[
        pltpu.VMEM((S, BPW), jnp.int32),
        pltpu.VMEM((2, BPW, D), jnp.float32),
        pltpu.VMEM((2, D, BPW), jnp.float32),
        pltpu.VMEM((S, D), jnp.float32),
        pltpu.SemaphoreType.DMA,
        pltpu.SemaphoreType.DMA,
        pltpu.SemaphoreType.DMA,
        pltpu.SemaphoreType.DMA,
        pltpu.SemaphoreType.DMA,
    ],
)
def _lookup(xt_hbm, tokr_hbm, pos_hbm, out_hbm, idx_all, rows_v, stage_v,
            pos_v, ssem, gsem0, gsem1, osem0, osem1):
    wid = lax.axis_index("s") * NC + lax.axis_index("c")
    b0 = wid * BPW
    gsems = (gsem0, gsem1)
    osems = (osem0, osem1)
    iota = lax.broadcasted_iota(jnp.int32, (16,), 0)

    cp_pos = pltpu.async_copy(pos_hbm, pos_v, ssem)
    cp_idx = pltpu.async_copy(xt_hbm.at[:, pl.ds(b0, BPW)], idx_all, ssem)
    cp_pos.wait()
    cp_idx.wait()

    def fire_gather(s, bufi, sem):
        pltpu.async_copy(tokr_hbm.at[idx_all.at[s]], rows_v.at[bufi], sem)

    def wait_gather(bufi, sem):
        pltpu.make_async_copy(
            tokr_hbm.at[pl.ds(0, BPW)], rows_v.at[bufi], sem,
        ).wait()

    def wait_out(bufi, sem):
        pltpu.make_async_copy(
            stage_v.at[bufi], out_hbm.at[0, pl.ds(0, D), pl.ds(0, BPW)], sem,
        ).wait()

    rowis = [lb * 16 + iota for lb in range(LB)]

    fire_gather(0, 0, gsem0)

    def pair(i, carry):
        for bufi in range(2):
            s = 2 * i + bufi
            wait_gather(bufi, gsems[bufi])

            @pl.when(s + 1 <= S - 1)
            def _():
                fire_gather(s + 1, 1 - bufi, gsems[1 - bufi])

            @pl.when(s >= 2)
            def _():
                wait_out(bufi, osems[bufi])

            def tok(j, inner):
                colj = jnp.full((16,), 0, jnp.int32) + j
                for lb in range(LB):
                    v = rows_v[bufi, j, pl.ds(lb * 16, 16)] \
                        + pos_v[s, pl.ds(lb * 16, 16)]
                    plsc.store_scatter(stage_v.at[bufi], [rowis[lb], colj], v)
                return inner

            lax.fori_loop(0, BPW, tok, 0)
            pltpu.async_copy(
                stage_v.at[bufi],
                out_hbm.at[s, pl.ds(0, D), pl.ds(b0, BPW)],
                osems[bufi],
            )
        return carry

    lax.fori_loop(0, S // 2, pair, 0)
    wait_out(0, osem0)
    wait_out(1, osem1)


def kernel(x, token_table, position_table):
    xt = x.astype(jnp.int32).T
    tokt = token_table.T
    tokr = _transpose_table(tokt)
    out = _lookup(xt, tokr, position_table)
    return jnp.transpose(out, (2, 0, 1))


# speedup vs baseline: 3.1354x; 3.1354x over previous
import functools

import jax
import jax.numpy as jnp
from jax import lax
from jax.experimental import pallas as pl
from jax.experimental.pallas import tpu as pltpu
from jax.experimental.pallas import tpu_sc as plsc

VOCAB = 1000000
S = 200
D = 64
B = 4096
NC = 2
NS = 16
NW = NC * NS
BPW = B // NW
LB = D // 16

CTOK = 128
NCHA = VOCAB // CTOK
TAIL = VOCAB - NCHA * CTOK
APW = 246
CG = CTOK // 16

_mesh = plsc.VectorSubcoreMesh(core_axis_name="c", subcore_axis_name="s")
_params = pltpu.CompilerParams(needs_layout_passes=False)


@functools.partial(
    pl.kernel,
    mesh=_mesh,
    compiler_params=_params,
    out_type=jax.ShapeDtypeStruct((VOCAB, 2 * D), jnp.float32),
    scratch_types=[
        pltpu.VMEM((2, D, CTOK), jnp.float32),
        pltpu.VMEM((2, CTOK, 2 * D), jnp.float32),
        pltpu.SemaphoreType.DMA,
        pltpu.SemaphoreType.DMA,
        pltpu.SemaphoreType.DMA,
        pltpu.SemaphoreType.DMA,
    ],
)
def _transpose_table(tokt_hbm, tail_hbm, tokr_hbm, in_v, out_v, isem0, isem1,
                     osem0, osem1):
    wid = lax.axis_index("s") * NC + lax.axis_index("c")
    isems = (isem0, isem1)
    osems = (osem0, osem1)
    iota = lax.broadcasted_iota(jnp.int32, (16,), 0)

    def chunk_id(slot):
        return wid + slot * NW

    def fire_in(slot, bufi, sem):
        pltpu.async_copy(
            tokt_hbm.at[:, pl.ds(chunk_id(slot) * CTOK, CTOK)],
            in_v.at[bufi], sem,
        )

    def wait_in(bufi, sem):
        pltpu.make_async_copy(
            tokt_hbm.at[:, pl.ds(0, CTOK)], in_v.at[bufi], sem,
        ).wait()

    def wait_out(bufi, sem):
        pltpu.make_async_copy(
            out_v.at[bufi], tokr_hbm.at[pl.ds(0, CTOK)], sem,
        ).wait()

    @pl.when(wid == 0)
    def _():
        pltpu.sync_copy(tail_hbm, tokr_hbm.at[pl.ds(NCHA * CTOK, TAIL)])

    fire_in(0, 0, isem0)

    def pair(i, carry):
        for bufi in range(2):
            slot = 2 * i + bufi

            @pl.when(chunk_id(slot) < NCHA)
            def _():
                wait_in(bufi, isems[bufi])

                @pl.when(chunk_id(slot + 1) < NCHA)
                def _():
                    fire_in(slot + 1, 1 - bufi, isems[1 - bufi])

                @pl.when(slot >= 2)
                def _():
                    wait_out(bufi, osems[bufi])

                def tr(g, inner):
                    rowi = g * 16 + iota
                    for dl in range(D):
                        plsc.store_scatter(
                            out_v.at[bufi],
                            [rowi, jnp.full((16,), dl, jnp.int32)],
                            in_v[bufi, dl, pl.ds(g * 16, 16)],
                        )
                    return inner

                lax.fori_loop(0, CG, tr, 0)
                pltpu.async_copy(
                    out_v.at[bufi],
                    tokr_hbm.at[pl.ds(chunk_id(slot) * CTOK, CTOK)],
                    osems[bufi],
                )
        return carry

    lax.fori_loop(0, APW // 2, pair, 0)
    wait_out(0, osem0)
    wait_out(1, osem1)


@functools.partial(
    pl.kernel,
    mesh=_mesh,
    compiler_params=_params,
    out_type=jax.ShapeDtypeStruct((S, D, B), jnp.float32),
    scratch_types=[
        pltpu.VMEM((S, BPW), jnp.int32),
        pltpu.VMEM((2, BPW, 2 * D), jnp.float32),
        pltpu.VMEM((2, D, BPW), jnp.float32),
        pltpu.VMEM((S * D,), jnp.float32),
        pltpu.SemaphoreType.DMA,
        pltpu.SemaphoreType.DMA,
        pltpu.SemaphoreType.DMA,
        pltpu.SemaphoreType.DMA,
        pltpu.SemaphoreType.DMA,
    ],
)
def _lookup(xt_hbm, tokr_hbm, pos_hbm, out_hbm, idx_all, rows_v, stage_v,
            pos_v, ssem, gsem0, gsem1, osem0, osem1):
    wid = lax.axis_index("s") * NC + lax.axis_index("c")
    b0 = wid * BPW
    gsems = (gsem0, gsem1)
    osems = (osem0, osem1)
    iota = lax.broadcasted_iota(jnp.int32, (16,), 0)

    cp_pos = pltpu.async_copy(pos_hbm, pos_v, ssem)
    cp_idx = pltpu.async_copy(xt_hbm.at[:, pl.ds(b0, BPW)], idx_all, ssem)
    cp_pos.wait()
    cp_idx.wait()

    def fire_gather(s, bufi, sem):
        pltpu.async_copy(tokr_hbm.at[idx_all.at[s]], rows_v.at[bufi], sem)

    def wait_gather(bufi, sem):
        pltpu.make_async_copy(
            tokr_hbm.at[pl.ds(0, BPW)], rows_v.at[bufi], sem,
        ).wait()

    def wait_out(bufi, sem):
        pltpu.make_async_copy(
            stage_v.at[bufi], out_hbm.at[0, pl.ds(0, D), pl.ds(0, BPW)], sem,
        ).wait()

    rowis = [lb * 16 + iota for lb in range(LB)]

    fire_gather(0, 0, gsem0)

    def pair(i, carry):
        for bufi in range(2):
            s = 2 * i + bufi
            wait_gather(bufi, gsems[bufi])

            @pl.when(s + 1 <= S - 1)
            def _():
                fire_gather(s + 1, 1 - bufi, gsems[1 - bufi])

            @pl.when(s >= 2)
            def _():
                wait_out(bufi, osems[bufi])

            pvs = [pos_v[pl.ds(s * D + lb * 16, 16)] for lb in range(LB)]

            def tok4(j4, inner):
                j = j4 * 4
                vals = []
                for k in range(4):
                    for lb in range(LB):
                        vals.append(
                            rows_v[bufi, j + k, pl.ds(lb * 16, 16)] + pvs[lb]
                        )
                for k in range(4):
                    colj = jnp.full((16,), 0, jnp.int32) + (j + k)
                    for lb in range(LB):
                        plsc.store_scatter(
                            stage_v.at[bufi], [rowis[lb], colj],
                            vals[k * LB + lb],
                        )
                return inner

            lax.fori_loop(0, BPW // 4, tok4, 0)
            pltpu.async_copy(
                stage_v.at[bufi],
                out_hbm.at[s, pl.ds(0, D), pl.ds(b0, BPW)],
                osems[bufi],
            )
        return carry

    lax.fori_loop(0, S // 2, pair, 0)
    wait_out(0, osem0)
    wait_out(1, osem1)


def kernel(x, token_table, position_table):
    xt = x.astype(jnp.int32).T
    tokt = token_table.T
    posf = position_table.reshape(-1)
    tail = jnp.pad(token_table[VOCAB - TAIL:, :], ((0, 0), (0, D)))
    tokr = _transpose_table(tokt, tail)
    out = _lookup(xt, tokr, posf)
    return jnp.transpose(out, (2, 0, 1))
